# baseline (speedup 1.0000x reference)
import functools

import jax
import jax.numpy as jnp
from jax import lax
from jax.experimental import pallas as pl
from jax.experimental.pallas import tpu as pltpu
from jax.experimental.pallas import tpu_sc as plsc

NC = 2
NS = 16
L = 16
NW = NC * NS
SDP = 16384
PRE = 512
IMAX = 2**31 - 1


def _phase_a_body(C, EPW, SP, n_comp,
                  ei_hbm, sd_hbm, e_hbm, cnt_hbm,
                  eiv, sdl, el, cv, sem):
    wid = lax.axis_index("s") * NC + lax.axis_index("c")
    E = ei_hbm.shape[1]
    base = jnp.where(wid == NW - 1, E - SP, wid * EPW)
    it16 = lax.iota(jnp.int32, L)

    pltpu.sync_copy(ei_hbm.at[:, pl.ds(base, SP)], eiv)

    def comp_body(c, off):
        s = eiv[0, pl.ds(c * L, L)]
        m = s < C

        def do_store():
            d = eiv[1, pl.ds(c * L, L)]
            e = base + c * L + it16
            plsc.store_compressed(sdl.at[pl.ds(off, L)], s * SDP + d, mask=m)
            plsc.store_compressed(el.at[pl.ds(off, L)], e, mask=m)
            return off + plsc.all_reduce_population_count(m)[0]

        return lax.cond(jnp.any(m), do_store, lambda: off)
    nv = lax.fori_loop(0, n_comp, comp_body, jnp.int32(0))

    cv[...] = jnp.full((L,), 1, jnp.int32) * nv
    pltpu.sync_copy(sdl, sd_hbm.at[wid])
    pltpu.sync_copy(el, e_hbm.at[wid])
    pltpu.sync_copy(cv, cnt_hbm.at[wid])


def _phase_b_body(N, C, SP, CPT,
                  sd_hbm, e_hbm, cnt_hbm, prob_hbm, x_hbm,
                  xrows_hbm, hasnb_hbm,
                  pb, sdb, eb, cntv, ovsd, ove, cp, ce, cd, cptr,
                  idxv, hv, rows, sem):
    wid = lax.axis_index("s") * NC + lax.axis_index("c")
    lo = wid * CPT
    it16 = lax.iota(jnp.int32, L)

    c1 = pltpu.async_copy(prob_hbm.at[pl.ds(lo * N, CPT * N)], pb, sem)
    c2 = pltpu.async_copy(sd_hbm.at[:, pl.ds(0, PRE)], sdb, sem)
    c3 = pltpu.async_copy(e_hbm.at[:, pl.ds(0, PRE)], eb, sem)
    c4 = pltpu.async_copy(cnt_hbm, cntv, sem)
    c1.wait()
    c2.wait()
    c3.wait()
    c4.wait()

    init = (jnp.full((L,), -1.0, jnp.float32),
            jnp.full((L,), 1, jnp.int32) * IMAX,
            jnp.zeros((L,), jnp.int32))

    def consume(sd, e, lm, best):
        bp, be, bd = best
        s = sd >> 14
        m = lm & (s >= lo) & (s < lo + CPT)

        def matched():
            cl = jnp.clip(s - lo, 0, CPT - 1)
            d = sd & (SDP - 1)
            p = plsc.load_gather(pb, [cl * N + jnp.clip(d, 0, N - 1)], mask=m)
            cp[...] = p
            ce[...] = e
            cd[...] = d
            cptr[...] = jnp.full((L,), -1, jnp.int32)

            def cond(act):
                return jnp.max(act) > 0

            def body(act):
                am = act > 0
                cur = plsc.load_gather(cptr, [cl], mask=am)
                hasc = am & (cur >= 0)
                safe = jnp.maximum(cur, 0)
                curp = plsc.load_gather(cp, [safe], mask=hasc)
                cure = plsc.load_gather(ce, [safe], mask=hasc)
                better = (cur < 0) | (p > curp) | ((p == curp) & (e < cure))
                nact = am & better
                plsc.store_scatter(cptr, [cl], it16, mask=nact)
                return nact.astype(jnp.int32)

            lax.while_loop(cond, body, m.astype(jnp.int32))

            ptr = cptr[...]
            mm = ptr >= 0
            safe2 = jnp.maximum(ptr, 0)
            fp = plsc.load_gather(cp, [safe2], mask=mm)
            fe = plsc.load_gather(ce, [safe2], mask=mm)
            fd = plsc.load_gather(cd, [safe2], mask=mm)
            b2 = mm & ((fp > bp) | ((fp == bp) & (fe < be)))
            return (jnp.where(b2, fp, bp), jnp.where(b2, fe, be),
                    jnp.where(b2, fd, bd))

        return lax.cond(jnp.any(m), matched, lambda: best)

    best = init
    for w in range(NW):
        cnt = cntv[w, pl.ds(0, L)][0]

        def k_pre(k, best, _w=w):
            sd = sdb[_w, pl.ds(k * L, L)]
            e = eb[_w, pl.ds(k * L, L)]
            lm = k * L + it16 < cnt
            return consume(sd, e, lm, best)

        best = lax.fori_loop(0, (jnp.minimum(cnt, PRE) + L - 1) // L,
                             k_pre, best)

        def q_body(q, best, _w=w):
            off = PRE + q * PRE
            pltpu.async_copy(sd_hbm.at[_w, pl.ds(off, PRE)], ovsd, sem).wait()
            pltpu.async_copy(e_hbm.at[_w, pl.ds(off, PRE)], ove, sem).wait()

            def k_ov(k, b):
                sd = ovsd[pl.ds(k * L, L)]
                e = ove[pl.ds(k * L, L)]
                lm = off + k * L + it16 < cnt
                return consume(sd, e, lm, b)

            rem = jnp.minimum(cnt - off, PRE)
            return lax.fori_loop(0, (rem + L - 1) // L, k_ov, best)

        best = lax.fori_loop(0, (jnp.maximum(cnt - PRE, 0) + PRE - 1) // PRE,
                             q_body, best)

    bp, be, bd = best
    has = bp >= 0.0
    idxv[...] = jnp.where(has, bd, 0)
    hv[...] = has.astype(jnp.int32)
    pltpu.async_copy(x_hbm.at[idxv], rows, sem).wait()
    pltpu.sync_copy(rows.at[pl.ds(0, CPT), :], xrows_hbm.at[pl.ds(lo, CPT)])
    pltpu.sync_copy(hv.at[pl.ds(0, CPT)], hasnb_hbm.at[pl.ds(lo, CPT)])


def _phase_c_body(xr_ref, w_ref, hn_ref, fb_ref, o_ref):
    xw = jnp.dot(xr_ref[...], w_ref[...], preferred_element_type=jnp.float32)
    act = jnp.where(xw > 0.0, xw, jnp.exp(xw) - 1.0)
    o_ref[...] = jnp.where(hn_ref[...] > 0, act, fb_ref[...])


@functools.lru_cache(maxsize=None)
def _build(N, E, C, IN_DIM, OUT_DIM):
    assert C % NW == 0 and C // NW <= L and N <= SDP
    CPT = C // NW
    SP = -(-(-(-E // NW)) // 128) * 128
    EPW = (E - SP) // (NW - 1) // 128 * 128
    assert EPW * (NW - 2) + SP >= E - SP and SP <= E and SP % PRE == 0
    n_comp = SP // L
    mesh = plsc.VectorSubcoreMesh(core_axis_name="c", subcore_axis_name="s",
                                  num_cores=NC, num_subcores=NS)

    pa = pl.kernel(
        functools.partial(_phase_a_body, C, EPW, SP, n_comp),
        out_type=(jax.ShapeDtypeStruct((NW, SP), jnp.int32),
                  jax.ShapeDtypeStruct((NW, SP), jnp.int32),
                  jax.ShapeDtypeStruct((NW, L), jnp.int32)),
        mesh=mesh,
        scratch_types=[
            pltpu.VMEM((2, SP), jnp.int32),
            pltpu.VMEM((SP,), jnp.int32),
            pltpu.VMEM((SP,), jnp.int32),
            pltpu.VMEM((L,), jnp.int32),
            pltpu.SemaphoreType.DMA,
        ],
        compiler_params=pltpu.CompilerParams(needs_layout_passes=False),
    )

    pb_ = pl.kernel(
        functools.partial(_phase_b_body, N, C, SP, CPT),
        out_type=(jax.ShapeDtypeStruct((C, IN_DIM), jnp.float32),
                  jax.ShapeDtypeStruct((C,), jnp.int32)),
        mesh=mesh,
        scratch_types=[
            pltpu.VMEM((CPT * N,), jnp.float32),
            pltpu.VMEM((NW, PRE), jnp.int32),
            pltpu.VMEM((NW, PRE), jnp.int32),
            pltpu.VMEM((NW, L), jnp.int32),
            pltpu.VMEM((PRE,), jnp.int32),
            pltpu.VMEM((PRE,), jnp.int32),
            pltpu.VMEM((L,), jnp.float32),
            pltpu.VMEM((L,), jnp.int32),
            pltpu.VMEM((L,), jnp.int32),
            pltpu.VMEM((L,), jnp.int32),
            pltpu.VMEM((L,), jnp.int32),
            pltpu.VMEM((L,), jnp.int32),
            pltpu.VMEM((L, IN_DIM), jnp.float32),
            pltpu.SemaphoreType.DMA,
        ],
        compiler_params=pltpu.CompilerParams(needs_layout_passes=False),
    )

    pc = pl.pallas_call(
        _phase_c_body,
        out_shape=jax.ShapeDtypeStruct((C, OUT_DIM), jnp.float32),
    )
    return pa, pb_, pc


def kernel(x, edge_index, edges_prob, class_idx, W, a):
    N, IN_DIM = x.shape
    E = edge_index.shape[1]
    C = class_idx.shape[0]
    OUT_DIM = W.shape[1]
    pa, pb_, pc = _build(N, E, C, IN_DIM, OUT_DIM)

    sd, el, cnt = pa(edge_index)
    prob_flat = edges_prob[:C].reshape(-1)
    xrows, hasnb = pb_(sd, el, cnt, prob_flat, x)

    rk = jax.random.key(1)
    h_rand = jax.vmap(
        lambda i: jax.random.normal(jax.random.fold_in(rk, i), (OUT_DIM,),
                                    dtype=jnp.float32))(class_idx)
    fallback = jax.nn.elu(h_rand)

    return pc(xrows, W, hasnb.reshape(C, 1), fallback)


# speedup vs baseline: 1.1111x; 1.1111x over previous
import functools

import jax
import jax.numpy as jnp
from jax import lax
from jax.experimental import pallas as pl
from jax.experimental.pallas import tpu as pltpu
from jax.experimental.pallas import tpu_sc as plsc

NC = 2
NS = 16
L = 16
NW = NC * NS
SDP = 16384
PRE = 512
IMAX = 2**31 - 1


def _phase_a_body(C, EPW, SP, n_comp,
                  ei_hbm, sd_hbm, e_hbm, sdp_hbm, ep_hbm, cnt_hbm,
                  eiv, sdl, el, cv, sem):
    wid = lax.axis_index("s") * NC + lax.axis_index("c")
    E = ei_hbm.shape[1]
    base = jnp.where(wid == NW - 1, E - SP, wid * EPW)
    it16 = lax.iota(jnp.int32, L)

    pltpu.sync_copy(ei_hbm.at[:, pl.ds(base, SP)], eiv)

    def comp_body(c, off):
        s = eiv[0, pl.ds(c * L, L)]
        m = s < C

        def do_store():
            d = eiv[1, pl.ds(c * L, L)]
            e = base + c * L + it16
            plsc.store_compressed(sdl.at[pl.ds(off, L)], s * SDP + d, mask=m)
            plsc.store_compressed(el.at[pl.ds(off, L)], e, mask=m)
            return off + plsc.all_reduce_population_count(m)[0]

        return lax.cond(jnp.any(m), do_store, lambda: off)
    nv = lax.fori_loop(0, n_comp, comp_body, jnp.int32(0))

    cv[...] = jnp.full((L,), 1, jnp.int32) * nv
    pltpu.sync_copy(sdl.at[pl.ds(0, PRE)], sdp_hbm.at[pl.ds(wid * PRE, PRE)])
    pltpu.sync_copy(el.at[pl.ds(0, PRE)], ep_hbm.at[pl.ds(wid * PRE, PRE)])
    pltpu.sync_copy(sdl, sd_hbm.at[wid])
    pltpu.sync_copy(el, e_hbm.at[wid])
    pltpu.sync_copy(cv, cnt_hbm.at[pl.ds(wid * L, L)])


def _phase_b_body(N, C, SP, CPT,
                  sd_hbm, e_hbm, sdp_hbm, ep_hbm, cnt_hbm, prob_hbm, x_hbm,
                  xrows_hbm, hasnb_hbm,
                  pb, sdb, eb, cntv, ovsd, ove, cp, ce, cd, cptr,
                  idxv, hv, rows, sem):
    wid = lax.axis_index("s") * NC + lax.axis_index("c")
    lo = wid * CPT
    it16 = lax.iota(jnp.int32, L)

    c1 = pltpu.async_copy(prob_hbm.at[pl.ds(lo * N, CPT * N)], pb, sem)
    c2 = pltpu.async_copy(sdp_hbm, sdb, sem)
    c3 = pltpu.async_copy(ep_hbm, eb, sem)
    c4 = pltpu.async_copy(cnt_hbm, cntv, sem)
    c1.wait()
    c2.wait()
    c3.wait()
    c4.wait()

    init = (jnp.full((L,), -1.0, jnp.float32),
            jnp.full((L,), 1, jnp.int32) * IMAX,
            jnp.zeros((L,), jnp.int32))

    def consume(sd, e, lm, best):
        bp, be, bd = best
        s = sd >> 14
        m = lm & (s >= lo) & (s < lo + CPT)

        def matched():
            cl = jnp.clip(s - lo, 0, CPT - 1)
            d = sd & (SDP - 1)
            p = plsc.load_gather(pb, [cl * N + jnp.clip(d, 0, N - 1)], mask=m)
            cp[...] = p
            ce[...] = e
            cd[...] = d
            cptr[...] = jnp.full((L,), -1, jnp.int32)

            def cond(act):
                return jnp.max(act) > 0

            def body(act):
                am = act > 0
                cur = plsc.load_gather(cptr, [cl], mask=am)
                hasc = am & (cur >= 0)
                safe = jnp.maximum(cur, 0)
                curp = plsc.load_gather(cp, [safe], mask=hasc)
                cure = plsc.load_gather(ce, [safe], mask=hasc)
                better = (cur < 0) | (p > curp) | ((p == curp) & (e < cure))
                nact = am & better
                plsc.store_scatter(cptr, [cl], it16, mask=nact)
                return nact.astype(jnp.int32)

            lax.while_loop(cond, body, m.astype(jnp.int32))

            ptr = cptr[...]
            mm = ptr >= 0
            safe2 = jnp.maximum(ptr, 0)
            fp = plsc.load_gather(cp, [safe2], mask=mm)
            fe = plsc.load_gather(ce, [safe2], mask=mm)
            fd = plsc.load_gather(cd, [safe2], mask=mm)
            b2 = mm & ((fp > bp) | ((fp == bp) & (fe < be)))
            return (jnp.where(b2, fp, bp), jnp.where(b2, fe, be),
                    jnp.where(b2, fd, bd))

        return lax.cond(jnp.any(m), matched, lambda: best)

    def w_body(w, best):
        cnt = cntv[pl.ds(w * L, L)][0]

        def k_pre(k, b):
            sd = sdb[pl.ds(w * PRE + k * L, L)]
            e = eb[pl.ds(w * PRE + k * L, L)]
            lm = k * L + it16 < cnt
            return consume(sd, e, lm, b)

        best = lax.fori_loop(0, (jnp.minimum(cnt, PRE) + L - 1) // L,
                             k_pre, best)

        def q_body(q, b):
            off = PRE + q * PRE
            pltpu.async_copy(sd_hbm.at[w, pl.ds(off, PRE)], ovsd, sem).wait()
            pltpu.async_copy(e_hbm.at[w, pl.ds(off, PRE)], ove, sem).wait()

            def k_ov(k, b2):
                sd = ovsd[pl.ds(k * L, L)]
                e = ove[pl.ds(k * L, L)]
                lm = off + k * L + it16 < cnt
                return consume(sd, e, lm, b2)

            rem = jnp.minimum(cnt - off, PRE)
            return lax.fori_loop(0, (rem + L - 1) // L, k_ov, b)

        return lax.fori_loop(0, (jnp.maximum(cnt - PRE, 0) + PRE - 1) // PRE,
                             q_body, best)

    best = lax.fori_loop(0, NW, w_body, init)

    bp, be, bd = best
    has = bp >= 0.0
    idxv[...] = jnp.where(has, bd, 0)
    hv[...] = has.astype(jnp.int32)
    pltpu.async_copy(x_hbm.at[idxv], rows, sem).wait()
    pltpu.sync_copy(rows.at[pl.ds(0, CPT), :], xrows_hbm.at[pl.ds(lo, CPT)])
    pltpu.sync_copy(hv.at[pl.ds(0, CPT)], hasnb_hbm.at[pl.ds(lo, CPT)])


def _phase_c_body(xr_ref, w_ref, hn_ref, fb_ref, o_ref):
    xw = jnp.dot(xr_ref[...], w_ref[...], preferred_element_type=jnp.float32)
    act = jnp.where(xw > 0.0, xw, jnp.exp(xw) - 1.0)
    o_ref[...] = jnp.where(hn_ref[...] > 0, act, fb_ref[...])


@functools.lru_cache(maxsize=None)
def _build(N, E, C, IN_DIM, OUT_DIM):
    assert C % NW == 0 and C // NW <= L and N <= SDP
    CPT = C // NW
    SP = -(-(-(-E // NW)) // 128) * 128
    EPW = (E - SP) // (NW - 1) // 128 * 128
    assert EPW * (NW - 2) + SP >= E - SP and SP <= E and SP % PRE == 0
    n_comp = SP // L
    mesh = plsc.VectorSubcoreMesh(core_axis_name="c", subcore_axis_name="s",
                                  num_cores=NC, num_subcores=NS)

    pa = pl.kernel(
        functools.partial(_phase_a_body, C, EPW, SP, n_comp),
        out_type=(jax.ShapeDtypeStruct((NW, SP), jnp.int32),
                  jax.ShapeDtypeStruct((NW, SP), jnp.int32),
                  jax.ShapeDtypeStruct((NW * PRE,), jnp.int32),
                  jax.ShapeDtypeStruct((NW * PRE,), jnp.int32),
                  jax.ShapeDtypeStruct((NW * L,), jnp.int32)),
        mesh=mesh,
        scratch_types=[
            pltpu.VMEM((2, SP), jnp.int32),
            pltpu.VMEM((SP,), jnp.int32),
            pltpu.VMEM((SP,), jnp.int32),
            pltpu.VMEM((L,), jnp.int32),
            pltpu.SemaphoreType.DMA,
        ],
        compiler_params=pltpu.CompilerParams(needs_layout_passes=False),
    )

    pb_ = pl.kernel(
        functools.partial(_phase_b_body, N, C, SP, CPT),
        out_type=(jax.ShapeDtypeStruct((C, IN_DIM), jnp.float32),
                  jax.ShapeDtypeStruct((C,), jnp.int32)),
        mesh=mesh,
        scratch_types=[
            pltpu.VMEM((CPT * N,), jnp.float32),
            pltpu.VMEM((NW * PRE,), jnp.int32),
            pltpu.VMEM((NW * PRE,), jnp.int32),
            pltpu.VMEM((NW * L,), jnp.int32),
            pltpu.VMEM((PRE,), jnp.int32),
            pltpu.VMEM((PRE,), jnp.int32),
            pltpu.VMEM((L,), jnp.float32),
            pltpu.VMEM((L,), jnp.int32),
            pltpu.VMEM((L,), jnp.int32),
            pltpu.VMEM((L,), jnp.int32),
            pltpu.VMEM((L,), jnp.int32),
            pltpu.VMEM((L,), jnp.int32),
            pltpu.VMEM((L, IN_DIM), jnp.float32),
            pltpu.SemaphoreType.DMA,
        ],
        compiler_params=pltpu.CompilerParams(needs_layout_passes=False),
    )

    pc = pl.pallas_call(
        _phase_c_body,
        out_shape=jax.ShapeDtypeStruct((C, OUT_DIM), jnp.float32),
    )
    return pa, pb_, pc


def kernel(x, edge_index, edges_prob, class_idx, W, a):
    N, IN_DIM = x.shape
    E = edge_index.shape[1]
    C = class_idx.shape[0]
    OUT_DIM = W.shape[1]
    pa, pb_, pc = _build(N, E, C, IN_DIM, OUT_DIM)

    sd, el, sdp, ep, cnt = pa(edge_index)
    prob_flat = edges_prob[:C].reshape(-1)
    xrows, hasnb = pb_(sd, el, sdp, ep, cnt, prob_flat, x)

    rk = jax.random.key(1)
    h_rand = jax.vmap(
        lambda i: jax.random.normal(jax.random.fold_in(rk, i), (OUT_DIM,),
                                    dtype=jnp.float32))(class_idx)
    fallback = jax.nn.elu(h_rand)

    return pc(xrows, W, hasnb.reshape(C, 1), fallback)


# speedup vs baseline: 1.5048x; 1.3544x over previous
import functools

import jax
import jax.numpy as jnp
from jax import lax
from jax.experimental import pallas as pl
from jax.experimental.pallas import tpu as pltpu
from jax.experimental.pallas import tpu_sc as plsc

NC = 2
NS = 16
L = 16
NW = NC * NS
GCH = 128
IMAX = 2**31 - 1


def _phase1_body(N, C, EPW, SP, CAP, n_comp, n_zero,
                 ei_hbm, prob_hbm, partp_hbm, parte_hbm, partd_hbm,
    eiv, linb, elv, probv, bptr, lp, le, ld,
                 sem):
    wid = lax.axis_index("s") * NC + lax.axis_index("c")
    E = ei_hbm.shape[1]
    base = jnp.where(wid == NW - 1, E - SP, wid * EPW)
    it16 = lax.iota(jnp.int32, L)
    z16 = jnp.zeros((L,), jnp.int32)
    o16 = jnp.full((L,), 1, jnp.int32)

    pltpu.sync_copy(ei_hbm.at[:, pl.ds(base, SP)], eiv)

    def comp_body(c, off):
        s = eiv[0, pl.ds(c * L, L)]
        m = s < C

        def do_store():
            e = base + c * L + it16
            plsc.store_compressed(elv.at[pl.ds(off, L)], e, mask=m)
            return off + plsc.all_reduce_population_count(m)[0]

        return lax.cond(jnp.any(m), do_store, lambda: off)
    nv = lax.fori_loop(0, n_comp, comp_body, jnp.int32(0))

    def g_body(g, _):
        for i in range(GCH // L):
            e16 = elv[pl.ds(g * GCH + i * L, L)]
            eloc = jnp.clip(e16 - base, 0, SP - 1)
            s = plsc.load_gather(eiv, [z16, eloc])
            d = plsc.load_gather(eiv, [o16, eloc])
            linb[pl.ds(i * L, L)] = jnp.minimum(s, C - 1) * N + d
        pltpu.async_copy(prob_hbm.at[linb],
                         probv.at[pl.ds(g * GCH, GCH)], sem).wait()
        return 0
    lax.fori_loop(0, (nv + GCH - 1) // GCH, g_body, 0)

    def init_body(i, _):
        bptr[pl.ds(i * L, L)] = jnp.full((L,), -1, jnp.int32)
        return 0
    lax.fori_loop(0, C // L, init_body, 0)

    def k_body(k, _):
        j = k * L + it16
        lm = j < nv
        e = elv[pl.ds(k * L, L)]
        p = probv[pl.ds(k * L, L)]
        eloc = jnp.clip(e - base, 0, SP - 1)
        c = jnp.minimum(plsc.load_gather(eiv, [z16, eloc]), C - 1)

        def cond(act):
            return jnp.max(act) > 0

        def body(act):
            am = act > 0
            cur = plsc.load_gather(bptr, [c], mask=am)
            hasc = am & (cur >= 0)
            safe = jnp.maximum(cur, 0)
            curp = plsc.load_gather(probv, [safe], mask=hasc)
            cure = plsc.load_gather(elv, [safe], mask=hasc)
            better = (cur < 0) | (p > curp) | ((p == curp) & (e < cure))
            nact = am & better
            plsc.store_scatter(bptr, [c], j, mask=nact)
            return nact.astype(jnp.int32)

        lax.while_loop(cond, body, lm.astype(jnp.int32))
        return 0
    lax.fori_loop(0, (nv + L - 1) // L, k_body, 0)

    def f_body(i, _):
        ptr = bptr[pl.ds(i * L, L)]
        m = ptr >= 0
        safe = jnp.maximum(ptr, 0)
        pv = plsc.load_gather(probv, [safe], mask=m)
        ev = plsc.load_gather(elv, [safe], mask=m)
        eloc = jnp.clip(ev - base, 0, SP - 1)
        dv = plsc.load_gather(eiv, [o16, eloc], mask=m)
        lp[pl.ds(i * L, L)] = jnp.where(m, pv, jnp.float32(-1.0))
        le[pl.ds(i * L, L)] = jnp.where(m, ev, IMAX)
        ld[pl.ds(i * L, L)] = jnp.where(m, dv, 0)
        return 0
    lax.fori_loop(0, C // L, f_body, 0)

    w1 = pltpu.async_copy(lp, partp_hbm.at[wid], sem)
    w2 = pltpu.async_copy(le, parte_hbm.at[wid], sem)
    w3 = pltpu.async_copy(ld, partd_hbm.at[wid], sem)
    w1.wait()
    w2.wait()
    w3.wait()


def _phase2_body(C, partp_hbm, parte_hbm, partd_hbm, x_hbm,
                 xrows_hbm, hasnb_hbm, pp, pe, pd, idxv, hv, rows, sem):
    wid = lax.axis_index("s") * NC + lax.axis_index("c")

    @pl.when(wid < C // L)
    def _():
        w1 = pltpu.async_copy(partp_hbm, pp, sem)
        w2 = pltpu.async_copy(parte_hbm, pe, sem)
        w3 = pltpu.async_copy(partd_hbm, pd, sem)
        w1.wait()
        w2.wait()
        w3.wait()
        colbase = wid * L

        bp = jnp.full((L,), -2.0, jnp.float32)
        be = jnp.zeros((L,), jnp.int32)
        bd = jnp.zeros((L,), jnp.int32)
        for r in range(NW):
            pv = pp[r, pl.ds(colbase, L)]
            ev = pe[r, pl.ds(colbase, L)]
            dv = pd[r, pl.ds(colbase, L)]
            better = (pv > bp) | ((pv == bp) & (ev < be))
            bp = jnp.where(better, pv, bp)
            be = jnp.where(better, ev, be)
            bd = jnp.where(better, dv, bd)

        has = bp >= 0.0
        idxv[...] = jnp.where(has, bd, 0)
        hv[...] = has.astype(jnp.int32)
        pltpu.async_copy(x_hbm.at[idxv], rows, sem).wait()
        pltpu.sync_copy(rows, xrows_hbm.at[pl.ds(colbase, L)])
        pltpu.sync_copy(hv, hasnb_hbm.at[pl.ds(colbase, L)])


def _phase3_body(xr_ref, w_ref, hn_ref, fb_ref, o_ref):
    xw = jnp.dot(xr_ref[...], w_ref[...], preferred_element_type=jnp.float32)
    act = jnp.where(xw > 0.0, xw, jnp.exp(xw) - 1.0)
    o_ref[...] = jnp.where(hn_ref[...] > 0, act, fb_ref[...])


@functools.lru_cache(maxsize=None)
def _build(N, E, C, IN_DIM, OUT_DIM):
    assert C % L == 0
    SP = -(-(-(-E // NW)) // 128) * 128
    EPW = (E - SP) // (NW - 1) // 128 * 128
    assert EPW * (NW - 2) + SP >= E - SP and SP <= E
    n_comp = SP // L
    CAP = SP
    n_zero = 0
    mesh = plsc.VectorSubcoreMesh(core_axis_name="c", subcore_axis_name="s",
                                  num_cores=NC, num_subcores=NS)

    p1 = pl.kernel(
        functools.partial(_phase1_body, N, C, EPW, SP, CAP, n_comp, n_zero),
        out_type=(jax.ShapeDtypeStruct((NW, C), jnp.float32),
                  jax.ShapeDtypeStruct((NW, C), jnp.int32),
                  jax.ShapeDtypeStruct((NW, C), jnp.int32)),
        mesh=mesh,
        scratch_types=[
            pltpu.VMEM((2, SP), jnp.int32),
            pltpu.VMEM((GCH,), jnp.int32),
            pltpu.VMEM((CAP,), jnp.int32),
            pltpu.VMEM((CAP,), jnp.float32),
            pltpu.VMEM((C,), jnp.int32),
            pltpu.VMEM((C,), jnp.float32),
            pltpu.VMEM((C,), jnp.int32),
            pltpu.VMEM((C,), jnp.int32),
            pltpu.SemaphoreType.DMA,
        ],
        compiler_params=pltpu.CompilerParams(needs_layout_passes=False),
    )

    p2 = pl.kernel(
        functools.partial(_phase2_body, C),
        out_type=(jax.ShapeDtypeStruct((C, IN_DIM), jnp.float32),
                  jax.ShapeDtypeStruct((C,), jnp.int32)),
        mesh=mesh,
        scratch_types=[
            pltpu.VMEM((NW, C), jnp.float32),
            pltpu.VMEM((NW, C), jnp.int32),
            pltpu.VMEM((NW, C), jnp.int32),
            pltpu.VMEM((L,), jnp.int32),
            pltpu.VMEM((L,), jnp.int32),
            pltpu.VMEM((L, IN_DIM), jnp.float32),
            pltpu.SemaphoreType.DMA,
        ],
        compiler_params=pltpu.CompilerParams(needs_layout_passes=False),
    )

    p3 = pl.pallas_call(
        _phase3_body,
        out_shape=jax.ShapeDtypeStruct((C, OUT_DIM), jnp.float32),
    )
    return p1, p2, p3


def kernel(x, edge_index, edges_prob, class_idx, W, a):
    N, IN_DIM = x.shape
    E = edge_index.shape[1]
    C = class_idx.shape[0]
    OUT_DIM = W.shape[1]
    p1, p2, p3 = _build(N, E, C, IN_DIM, OUT_DIM)

    prob_flat = edges_prob[:C].reshape(-1)
    partp, parte, partd = p1(edge_index, prob_flat)
    xrows, hasnb = p2(partp, parte, partd, x)

    rk = jax.random.key(1)
    h_rand = jax.vmap(
        lambda i: jax.random.normal(jax.random.fold_in(rk, i), (OUT_DIM,),
                                    dtype=jnp.float32))(class_idx)
    fallback = jax.nn.elu(h_rand)

    return p3(xrows, W, hasnb.reshape(C, 1), fallback)


# speedup vs baseline: 1.5766x; 1.0477x over previous
import functools

import jax
import jax.numpy as jnp
from jax import lax
from jax.experimental import pallas as pl
from jax.experimental.pallas import tpu as pltpu
from jax.experimental.pallas import tpu_sc as plsc

NC = 2
NS = 16
L = 16
NW = NC * NS
GCH = 128
IMAX = 2**31 - 1


def _phase1_body(N, C, EPW, SP, CAP, n_comp, n_zero,
                 ei_hbm, prob_hbm, partp_hbm, parte_hbm, partd_hbm,
    eiv, linb, elv, probv, bptr, lp, le, ld,
                 sem):
    wid = lax.axis_index("s") * NC + lax.axis_index("c")
    E = ei_hbm.shape[1]
    base = jnp.where(wid == NW - 1, E - SP, wid * EPW)
    it16 = lax.iota(jnp.int32, L)
    z16 = jnp.zeros((L,), jnp.int32)
    o16 = jnp.full((L,), 1, jnp.int32)

    pltpu.sync_copy(ei_hbm.at[:, pl.ds(base, SP)], eiv)

    def comp_body(c, off):
        s = eiv[0, pl.ds(c * L, L)]
        m = s < C
        e = base + c * L + it16
        plsc.store_compressed(elv.at[pl.ds(off, L)], e, mask=m)
        return off + plsc.all_reduce_population_count(m)[0]
    nv = lax.fori_loop(0, n_comp, comp_body, jnp.int32(0))

    def g_body(g, _):
        for i in range(GCH // L):
            e16 = elv[pl.ds(g * GCH + i * L, L)]
            eloc = jnp.clip(e16 - base, 0, SP - 1)
            s = plsc.load_gather(eiv, [z16, eloc])
            d = plsc.load_gather(eiv, [o16, eloc])
            linb[pl.ds(i * L, L)] = jnp.minimum(s, C - 1) * N + d
        pltpu.async_copy(prob_hbm.at[linb],
                         probv.at[pl.ds(g * GCH, GCH)], sem).wait()
        return 0
    lax.fori_loop(0, (nv + GCH - 1) // GCH, g_body, 0)

    def init_body(i, _):
        bptr[pl.ds(i * L, L)] = jnp.full((L,), -1, jnp.int32)
        return 0
    lax.fori_loop(0, C // L, init_body, 0)

    def k_body(k, _):
        j = k * L + it16
        lm = j < nv
        e = elv[pl.ds(k * L, L)]
        p = probv[pl.ds(k * L, L)]
        eloc = jnp.clip(e - base, 0, SP - 1)
        c = jnp.minimum(plsc.load_gather(eiv, [z16, eloc]), C - 1)

        def cond(act):
            return jnp.max(act) > 0

        def body(act):
            am = act > 0
            cur = plsc.load_gather(bptr, [c], mask=am)
            hasc = am & (cur >= 0)
            safe = jnp.maximum(cur, 0)
            curp = plsc.load_gather(probv, [safe], mask=hasc)
            cure = plsc.load_gather(elv, [safe], mask=hasc)
            better = (cur < 0) | (p > curp) | ((p == curp) & (e < cure))
            nact = am & better
            plsc.store_scatter(bptr, [c], j, mask=nact)
            return nact.astype(jnp.int32)

        lax.while_loop(cond, body, lm.astype(jnp.int32))
        return 0
    lax.fori_loop(0, (nv + L - 1) // L, k_body, 0)

    def f_body(i, _):
        ptr = bptr[pl.ds(i * L, L)]
        m = ptr >= 0
        safe = jnp.maximum(ptr, 0)
        pv = plsc.load_gather(probv, [safe], mask=m)
        ev = plsc.load_gather(elv, [safe], mask=m)
        eloc = jnp.clip(ev - base, 0, SP - 1)
        dv = plsc.load_gather(eiv, [o16, eloc], mask=m)
        lp[pl.ds(i * L, L)] = jnp.where(m, pv, jnp.float32(-1.0))
        le[pl.ds(i * L, L)] = jnp.where(m, ev, IMAX)
        ld[pl.ds(i * L, L)] = jnp.where(m, dv, 0)
        return 0
    lax.fori_loop(0, C // L, f_body, 0)

    w1 = pltpu.async_copy(lp, partp_hbm.at[wid], sem)
    w2 = pltpu.async_copy(le, parte_hbm.at[wid], sem)
    w3 = pltpu.async_copy(ld, partd_hbm.at[wid], sem)
    w1.wait()
    w2.wait()
    w3.wait()


def _phase2_body(C, partp_hbm, parte_hbm, partd_hbm, x_hbm,
                 xrows_hbm, hasnb_hbm, pp, pe, pd, idxv, hv, rows, sem):
    wid = lax.axis_index("s") * NC + lax.axis_index("c")

    @pl.when(wid < C // L)
    def _():
        w1 = pltpu.async_copy(partp_hbm, pp, sem)
        w2 = pltpu.async_copy(parte_hbm, pe, sem)
        w3 = pltpu.async_copy(partd_hbm, pd, sem)
        w1.wait()
        w2.wait()
        w3.wait()
        colbase = wid * L

        bp = jnp.full((L,), -2.0, jnp.float32)
        be = jnp.zeros((L,), jnp.int32)
        bd = jnp.zeros((L,), jnp.int32)
        for r in range(NW):
            pv = pp[r, pl.ds(colbase, L)]
            ev = pe[r, pl.ds(colbase, L)]
            dv = pd[r, pl.ds(colbase, L)]
            better = (pv > bp) | ((pv == bp) & (ev < be))
            bp = jnp.where(better, pv, bp)
            be = jnp.where(better, ev, be)
            bd = jnp.where(better, dv, bd)

        has = bp >= 0.0
        idxv[...] = jnp.where(has, bd, 0)
        hv[...] = has.astype(jnp.int32)
        pltpu.async_copy(x_hbm.at[idxv], rows, sem).wait()
        pltpu.sync_copy(rows, xrows_hbm.at[pl.ds(colbase, L)])
        pltpu.sync_copy(hv, hasnb_hbm.at[pl.ds(colbase, L)])


def _phase3_body(xr_ref, w_ref, hn_ref, fb_ref, o_ref):
    xw = jnp.dot(xr_ref[...], w_ref[...], preferred_element_type=jnp.float32)
    act = jnp.where(xw > 0.0, xw, jnp.exp(xw) - 1.0)
    o_ref[...] = jnp.where(hn_ref[...] > 0, act, fb_ref[...])


@functools.lru_cache(maxsize=None)
def _build(N, E, C, IN_DIM, OUT_DIM):
    assert C % L == 0
    SP = -(-(-(-E // NW)) // 128) * 128
    EPW = (E - SP) // (NW - 1) // 128 * 128
    assert EPW * (NW - 2) + SP >= E - SP and SP <= E
    n_comp = SP // L
    CAP = SP
    n_zero = 0
    mesh = plsc.VectorSubcoreMesh(core_axis_name="c", subcore_axis_name="s",
                                  num_cores=NC, num_subcores=NS)

    p1 = pl.kernel(
        functools.partial(_phase1_body, N, C, EPW, SP, CAP, n_comp, n_zero),
        out_type=(jax.ShapeDtypeStruct((NW, C), jnp.float32),
                  jax.ShapeDtypeStruct((NW, C), jnp.int32),
                  jax.ShapeDtypeStruct((NW, C), jnp.int32)),
        mesh=mesh,
        scratch_types=[
            pltpu.VMEM((2, SP), jnp.int32),
            pltpu.VMEM((GCH,), jnp.int32),
            pltpu.VMEM((CAP,), jnp.int32),
            pltpu.VMEM((CAP,), jnp.float32),
            pltpu.VMEM((C,), jnp.int32),
            pltpu.VMEM((C,), jnp.float32),
            pltpu.VMEM((C,), jnp.int32),
            pltpu.VMEM((C,), jnp.int32),
            pltpu.SemaphoreType.DMA,
        ],
        compiler_params=pltpu.CompilerParams(needs_layout_passes=False),
    )

    p2 = pl.kernel(
        functools.partial(_phase2_body, C),
        out_type=(jax.ShapeDtypeStruct((C, IN_DIM), jnp.float32),
                  jax.ShapeDtypeStruct((C,), jnp.int32)),
        mesh=mesh,
        scratch_types=[
            pltpu.VMEM((NW, C), jnp.float32),
            pltpu.VMEM((NW, C), jnp.int32),
            pltpu.VMEM((NW, C), jnp.int32),
            pltpu.VMEM((L,), jnp.int32),
            pltpu.VMEM((L,), jnp.int32),
            pltpu.VMEM((L, IN_DIM), jnp.float32),
            pltpu.SemaphoreType.DMA,
        ],
        compiler_params=pltpu.CompilerParams(needs_layout_passes=False),
    )

    p3 = pl.pallas_call(
        _phase3_body,
        out_shape=jax.ShapeDtypeStruct((C, OUT_DIM), jnp.float32),
    )
    return p1, p2, p3


def kernel(x, edge_index, edges_prob, class_idx, W, a):
    N, IN_DIM = x.shape
    E = edge_index.shape[1]
    C = class_idx.shape[0]
    OUT_DIM = W.shape[1]
    p1, p2, p3 = _build(N, E, C, IN_DIM, OUT_DIM)

    prob_flat = edges_prob[:C].reshape(-1)
    partp, parte, partd = p1(edge_index, prob_flat)
    xrows, hasnb = p2(partp, parte, partd, x)

    rk = jax.random.key(1)
    h_rand = jax.vmap(
        lambda i: jax.random.normal(jax.random.fold_in(rk, i), (OUT_DIM,),
                                    dtype=jnp.float32))(class_idx)
    fallback = jax.nn.elu(h_rand)

    return p3(xrows, W, hasnb.reshape(C, 1), fallback)
